# baseline (speedup 1.0000x reference)
import jax
import jax.numpy as jnp
from jax.experimental import pallas as pl
from jax.experimental.pallas import tpu as pltpu

_NS = 256
_NH = 512
_TA = 128
_R2 = 256
_R3 = 768
_BM = 4096


def _mlp_kernel(x_ref, w1_ref, w2_ref, wf_ref, b1_ref, b2_ref, bf_ref, o_ref):
    zero = jnp.zeros((), jnp.bfloat16)
    x = x_ref[...].astype(jnp.bfloat16)
    h = jnp.dot(x, w1_ref[...], preferred_element_type=jnp.float32)
    h = jnp.maximum(h.astype(jnp.bfloat16) + b1_ref[...], zero)
    h = jnp.dot(h, w2_ref[...], preferred_element_type=jnp.float32)
    h = jnp.maximum(h.astype(jnp.bfloat16) + b2_ref[...], zero)
    o_ref[...] = jnp.tanh(
        jnp.dot(h, wf_ref[...], preferred_element_type=jnp.float32)
        + bf_ref[...])


def kernel(x, w_slab, b_slab):
    B = x.shape[0]
    w1 = w_slab[0:_NS, 0:_NH].astype(jnp.bfloat16)
    w2 = w_slab[_R2:_R2 + _NH, 0:_NH].astype(jnp.bfloat16)
    wf = w_slab[_R3:_R3 + _NH, 0:_TA].astype(jnp.bfloat16)
    b1 = b_slab[0:1, 0:_NH].astype(jnp.bfloat16)
    b2 = b_slab[1:2, 0:_NH].astype(jnp.bfloat16)
    bf = b_slab[2:3, 0:_TA]

    Bp = (B + _BM - 1) // _BM * _BM
    if Bp != B:
        x = jnp.pad(x, ((0, Bp - B), (0, 0)))

    out = pl.pallas_call(
        _mlp_kernel,
        out_shape=jax.ShapeDtypeStruct((Bp, _TA), jnp.float32),
        grid=(Bp // _BM,),
        in_specs=[
            pl.BlockSpec((_BM, _NS), lambda i: (i, 0)),
            pl.BlockSpec(w1.shape, lambda i: (0, 0)),
            pl.BlockSpec(w2.shape, lambda i: (0, 0)),
            pl.BlockSpec(wf.shape, lambda i: (0, 0)),
            pl.BlockSpec(b1.shape, lambda i: (0, 0)),
            pl.BlockSpec(b2.shape, lambda i: (0, 0)),
            pl.BlockSpec(bf.shape, lambda i: (0, 0)),
        ],
        out_specs=pl.BlockSpec((_BM, _TA), lambda i: (i, 0)),
        compiler_params=pltpu.CompilerParams(
            dimension_semantics=("parallel",)),
    )(x, w1, w2, wf, b1, b2, bf)
    return out[:B]


# speedup vs baseline: 1.0049x; 1.0049x over previous
import jax
import jax.numpy as jnp
from jax.experimental import pallas as pl
from jax.experimental.pallas import tpu as pltpu

_NS = 256
_NH = 512
_TA = 128
_R2 = 256
_R3 = 768
_BM = 4096


def _mlp_kernel(x_ref, w1_ref, w2_ref, wf_ref, b1_ref, b2_ref, bf_ref, o_ref):
    x = x_ref[...].astype(jnp.bfloat16)
    h = jnp.dot(x, w1_ref[...], preferred_element_type=jnp.float32) + b1_ref[...]
    h = jnp.maximum(h, 0.0).astype(jnp.bfloat16)
    h = jnp.dot(h, w2_ref[...], preferred_element_type=jnp.float32) + b2_ref[...]
    h = jnp.maximum(h, 0.0).astype(jnp.bfloat16)
    o_ref[...] = jnp.tanh(
        jnp.dot(h, wf_ref[...], preferred_element_type=jnp.float32)
        + bf_ref[...])


def kernel(x, w_slab, b_slab):
    B = x.shape[0]
    w1 = w_slab[0:_NS, 0:_NH].astype(jnp.bfloat16)
    w2 = w_slab[_R2:_R2 + _NH, 0:_NH].astype(jnp.bfloat16)
    wf = w_slab[_R3:_R3 + _NH, 0:_TA].astype(jnp.bfloat16)
    b1 = b_slab[0:1, 0:_NH]
    b2 = b_slab[1:2, 0:_NH]
    bf = b_slab[2:3, 0:_TA]

    Bp = (B + _BM - 1) // _BM * _BM
    if Bp != B:
        x = jnp.pad(x, ((0, Bp - B), (0, 0)))

    out = pl.pallas_call(
        _mlp_kernel,
        out_shape=jax.ShapeDtypeStruct((Bp, _TA), jnp.float32),
        grid=(Bp // _BM,),
        in_specs=[
            pl.BlockSpec((_BM, _NS), lambda i: (i, 0)),
            pl.BlockSpec(w1.shape, lambda i: (0, 0)),
            pl.BlockSpec(w2.shape, lambda i: (0, 0)),
            pl.BlockSpec(wf.shape, lambda i: (0, 0)),
            pl.BlockSpec(b1.shape, lambda i: (0, 0)),
            pl.BlockSpec(b2.shape, lambda i: (0, 0)),
            pl.BlockSpec(bf.shape, lambda i: (0, 0)),
        ],
        out_specs=pl.BlockSpec((_BM, _TA), lambda i: (i, 0)),
        compiler_params=pltpu.CompilerParams(
            dimension_semantics=("parallel",)),
    )(x, w1, w2, wf, b1, b2, bf)
    return out[:B]


# speedup vs baseline: 1.0590x; 1.0538x over previous
import jax
import jax.numpy as jnp
from jax.experimental import pallas as pl
from jax.experimental.pallas import tpu as pltpu

_NS = 256
_NH = 512
_TA = 128
_R2 = 256
_R3 = 768
_BM = 4096


def _mlp_kernel(x_ref, w1_ref, w2_ref, wf_ref, b_ref, o_ref):
    x = x_ref[...].astype(jnp.bfloat16)
    b1 = b_ref[0:1, 0:_NH]
    b2 = b_ref[1:2, 0:_NH]
    bf = b_ref[2:3, 0:_TA]
    h = jnp.dot(x, w1_ref[...], preferred_element_type=jnp.float32) + b1
    h = jnp.maximum(h, 0.0).astype(jnp.bfloat16)
    h = jnp.dot(h, w2_ref[...], preferred_element_type=jnp.float32) + b2
    h = jnp.maximum(h, 0.0).astype(jnp.bfloat16)
    o_ref[...] = jnp.tanh(
        jnp.dot(h, wf_ref[...], preferred_element_type=jnp.float32) + bf)


def kernel(x, w_slab, b_slab):
    B = x.shape[0]
    w1 = w_slab[0:_NS, 0:_NH].astype(jnp.bfloat16)
    w2 = w_slab[_R2:_R2 + _NH, 0:_NH].astype(jnp.bfloat16)
    wf = w_slab[_R3:_R3 + _NH, 0:_TA].astype(jnp.bfloat16)
    Bp = (B + _BM - 1) // _BM * _BM
    if Bp != B:
        x = jnp.pad(x, ((0, Bp - B), (0, 0)))

    out = pl.pallas_call(
        _mlp_kernel,
        out_shape=jax.ShapeDtypeStruct((Bp, _TA), jnp.float32),
        grid=(Bp // _BM,),
        in_specs=[
            pl.BlockSpec((_BM, _NS), lambda i: (i, 0)),
            pl.BlockSpec(w1.shape, lambda i: (0, 0)),
            pl.BlockSpec(w2.shape, lambda i: (0, 0)),
            pl.BlockSpec(wf.shape, lambda i: (0, 0)),
            pl.BlockSpec(b_slab.shape, lambda i: (0, 0)),
        ],
        out_specs=pl.BlockSpec((_BM, _TA), lambda i: (i, 0)),
        compiler_params=pltpu.CompilerParams(
            dimension_semantics=("parallel",)),
    )(x, w1, w2, wf, b_slab)
    return out[:B]


# speedup vs baseline: 1.0855x; 1.0250x over previous
import jax
import jax.numpy as jnp
from jax.experimental import pallas as pl
from jax.experimental.pallas import tpu as pltpu

_NS = 256
_NH = 512
_TA = 128
_R2 = 256
_R3 = 768
_BM = 4096


def _mlp_kernel(x_ref, w_ref, b_ref, o_ref):
    x = x_ref[...].astype(jnp.bfloat16)
    w1 = w_ref[0:_NS, 0:_NH]
    w2 = w_ref[_R2:_R2 + _NH, 0:_NH]
    wf = w_ref[_R3:_R3 + _NH, 0:_TA]
    b1 = b_ref[0:1, 0:_NH]
    b2 = b_ref[1:2, 0:_NH]
    bf = b_ref[2:3, 0:_TA]
    h = jnp.dot(x, w1, preferred_element_type=jnp.float32) + b1
    h = jnp.maximum(h, 0.0).astype(jnp.bfloat16)
    h = jnp.dot(h, w2, preferred_element_type=jnp.float32) + b2
    h = jnp.maximum(h, 0.0).astype(jnp.bfloat16)
    o_ref[...] = jnp.tanh(
        jnp.dot(h, wf, preferred_element_type=jnp.float32) + bf)


def kernel(x, w_slab, b_slab):
    B = x.shape[0]
    ws = w_slab.astype(jnp.bfloat16)
    Bp = (B + _BM - 1) // _BM * _BM
    if Bp != B:
        x = jnp.pad(x, ((0, Bp - B), (0, 0)))

    out = pl.pallas_call(
        _mlp_kernel,
        out_shape=jax.ShapeDtypeStruct((Bp, _TA), jnp.float32),
        grid=(Bp // _BM,),
        in_specs=[
            pl.BlockSpec((_BM, _NS), lambda i: (i, 0)),
            pl.BlockSpec(ws.shape, lambda i: (0, 0)),
            pl.BlockSpec(b_slab.shape, lambda i: (0, 0)),
        ],
        out_specs=pl.BlockSpec((_BM, _TA), lambda i: (i, 0)),
        compiler_params=pltpu.CompilerParams(
            dimension_semantics=("parallel",)),
    )(x, ws, b_slab)
    return out[:B]


# speedup vs baseline: 1.1544x; 1.0635x over previous
import jax
import jax.numpy as jnp
from jax.experimental import pallas as pl
from jax.experimental.pallas import tpu as pltpu

_NS = 256
_NH = 512
_TA = 128
_R2 = 256
_R3 = 768
_BM = 4096


def _mlp_kernel(x_ref, w_ref, b_ref, o_ref):
    x = x_ref[...].astype(jnp.bfloat16)
    w1 = w_ref[0:_NS, 0:_NH].astype(jnp.bfloat16)
    w2 = w_ref[_R2:_R2 + _NH, 0:_NH].astype(jnp.bfloat16)
    wf = w_ref[_R3:_R3 + _NH, 0:_TA].astype(jnp.bfloat16)
    b1 = b_ref[0:1, 0:_NH]
    b2 = b_ref[1:2, 0:_NH]
    bf = b_ref[2:3, 0:_TA]
    h = jnp.dot(x, w1, preferred_element_type=jnp.float32) + b1
    h = jnp.maximum(h, 0.0).astype(jnp.bfloat16)
    h = jnp.dot(h, w2, preferred_element_type=jnp.float32) + b2
    h = jnp.maximum(h, 0.0).astype(jnp.bfloat16)
    o_ref[...] = jnp.tanh(
        jnp.dot(h, wf, preferred_element_type=jnp.float32) + bf)


def kernel(x, w_slab, b_slab):
    B = x.shape[0]
    Bp = (B + _BM - 1) // _BM * _BM
    if Bp != B:
        x = jnp.pad(x, ((0, Bp - B), (0, 0)))

    out = pl.pallas_call(
        _mlp_kernel,
        out_shape=jax.ShapeDtypeStruct((Bp, _TA), jnp.float32),
        grid=(Bp // _BM,),
        in_specs=[
            pl.BlockSpec((_BM, _NS), lambda i: (i, 0)),
            pl.BlockSpec(w_slab.shape, lambda i: (0, 0)),
            pl.BlockSpec(b_slab.shape, lambda i: (0, 0)),
        ],
        out_specs=pl.BlockSpec((_BM, _TA), lambda i: (i, 0)),
        compiler_params=pltpu.CompilerParams(
            dimension_semantics=("parallel",)),
    )(x, w_slab, b_slab)
    return out[:B]
